# baseline (speedup 1.0000x reference)
import functools

import jax
import jax.numpy as jnp
import numpy as np
from jax import lax
from jax.experimental import pallas as pl
from jax.experimental.pallas import tpu as pltpu
from jax.experimental.pallas import tpu_sc as plsc

IRREPS = [(128, 0, 1), (64, 1, -1), (32, 2, 1)]
G = 256
EPS = 1e-05
DIM = 480
NMUL = 224
NSC = 128
SW = 640
TW = 640
BN = 1000
W = 40
GP = G + W

CHUNK = 200
NLANE = 16


GPT = G // 32


def _sc_stats_body(x_hbm, bnd_hbm, o_hbm, xbuf, bndv, local):
    cid = lax.axis_index("c")
    sid = lax.axis_index("s")
    wid = sid * 2 + cid

    pltpu.sync_copy(
        bnd_hbm.at[pl.ds(pl.multiple_of(wid * NLANE, 8), NLANE)], bndv)
    bounds = bndv[...]
    r0 = bounds[0]
    r1 = bounds[GPT]

    zero16 = jnp.zeros((NLANE,), jnp.float32)
    ones16 = jnp.ones((NLANE,), jnp.float32)
    for i in range(GPT):
        for j in range(SW // NLANE):
            local[i, pl.ds(j * NLANE, NLANE)] = zero16

    start0 = pl.multiple_of((r0 // 8) * 8, 8)
    rtail = pl.multiple_of(
        jnp.maximum((r1 - CHUNK + 7) // 8 * 8, 0), 8)
    nch = (r1 - start0 + CHUNK - 1) // CHUNK

    def do_chunk(k, pend):
        cb = pl.multiple_of(jnp.minimum(start0 + k * CHUNK, rtail), 8)
        hi_abs = jnp.minimum(cb + CHUNK, r1)
        pltpu.sync_copy(x_hbm.at[pl.ds(cb, CHUNK), :], xbuf)
        pstart = jnp.maximum(pend, cb)

        for g in range(GPT):
            lo = jnp.maximum(bounds[g] - cb, pstart - cb)
            hi = jnp.maximum(jnp.minimum(bounds[g + 1], hi_abs) - cb, lo)

            def row(r, _, g=g):
                for j in range(DIM // NLANE):
                    v = xbuf[r, pl.ds(j * NLANE, NLANE)]
                    if j < NSC // NLANE:
                        local[g, pl.ds(j * NLANE, NLANE)] += v
                    local[g, pl.ds(NSC + j * NLANE, NLANE)] += v * v
                return 0

            lax.fori_loop(lo, hi, row, 0)
        return jnp.maximum(hi_abs, pend)

    lax.fori_loop(0, nch, do_chunk, r0)
    for g in range(GPT):
        cnt = (bounds[g + 1] - bounds[g]).astype(jnp.float32)
        local[g, pl.ds(SW - 32, NLANE)] = jnp.full((NLANE,), cnt)
    pltpu.sync_copy(local,
                    o_hbm.at[pl.ds(pl.multiple_of(wid * GPT, 8), GPT), :])


def _sc_stats(x, bnd):
    mesh = plsc.VectorSubcoreMesh(core_axis_name="c", subcore_axis_name="s")
    f = pl.kernel(
        _sc_stats_body,
        out_type=jax.ShapeDtypeStruct((G, SW), jnp.float32),
        mesh=mesh,
        scratch_types=[
            pltpu.VMEM((CHUNK, DIM), jnp.float32),
            pltpu.VMEM((NLANE,), jnp.int32),
            pltpu.VMEM((GPT, SW), jnp.float32),
        ],
    )
    return f(x, bnd)


def _table_kernel(st_ref, gm_ref, em_ref, dinv_ref, w_ref, ms_ref,
                  bias_ref, t_ref):
    st = st_ref[...]
    cnt = jnp.maximum(st[:, SW - 32:SW - 31], 1.0)
    s1 = st[:, :NSC]
    sq = st[:, NSC:NSC + DIM]
    m = s1 / cnt
    gq = jax.lax.dot_general(sq, gm_ref[...], (((1,), (0,)), ((), ())),
                             preferred_element_type=jnp.float32)
    ex2 = gq * dinv_ref[...] / cnt
    ms = ms_ref[...]
    corr = (m * m) * (ms * (2.0 - ms))
    corr_p = jnp.concatenate([corr, jnp.zeros((G, NMUL - NSC), jnp.float32)],
                             axis=1)
    fn = jax.lax.rsqrt(ex2 - corr_p + EPS) * w_ref[...]
    scale = jax.lax.dot_general(fn, em_ref[...], (((1,), (0,)), ((), ())),
                                preferred_element_type=jnp.float32)
    off_sc = bias_ref[...] - m * ms * fn[:, :NSC]
    tbl = jnp.concatenate([scale, off_sc,
                           jnp.zeros((G, TW - DIM - NSC), jnp.float32)],
                          axis=1)
    t_ref[...] = jnp.concatenate(
        [tbl, jnp.zeros((GP - G, TW), jnp.float32)], axis=0)


def _apply_kernel(x_ref, b_ref, b0_ref, sp_ref, t_ref, o_ref):
    x = x_ref[...]
    bn = x.shape[0]
    seg = b_ref[0, 0, :]
    b0a = pl.multiple_of(b0_ref[pl.program_id(0)], 8)
    span = sp_ref[pl.program_id(0)]

    def fin(so):
        o_ref[...] = jnp.concatenate(
            [x[:, :NSC] * so[:, :NSC] + so[:, DIM:DIM + NSC],
             x[:, NSC:] * so[:, NSC:DIM]], axis=1)

    @pl.when(span <= W - 8)
    def _():
        iota = jax.lax.broadcasted_iota(jnp.int32, (BN, W), 1) + b0a
        oh = (iota == seg[:, None]).astype(jnp.float32)
        fin(jax.lax.dot_general(oh, t_ref[pl.ds(b0a, W), :],
                                (((1,), (0,)), ((), ())),
                                preferred_element_type=jnp.float32))

    @pl.when(span > W - 8)
    def _():
        iota = jax.lax.broadcasted_iota(jnp.int32, (BN, G), 1)
        oh = (iota == seg[:, None]).astype(jnp.float32)
        fin(jax.lax.dot_general(oh, t_ref[pl.ds(0, G), :],
                                (((1,), (0,)), ((), ())),
                                preferred_element_type=jnp.float32))


def _build_consts():
    d_of = np.concatenate([np.full(mul, 2 * l + 1, np.float32)
                           for mul, l, p in IRREPS])
    gm = np.zeros((DIM, NMUL), np.float32)
    em = np.zeros((NMUL, DIM), np.float32)
    c = 0
    mi = 0
    for mul, l, p in IRREPS:
        d = 2 * l + 1
        for k in range(mul):
            gm[c:c + d, mi] = 1.0
            em[mi, c:c + d] = 1.0
            c += d
            mi += 1
    dinv = (1.0 / d_of)[None, :]
    return gm, em, dinv


_GM, _EM, _DINV = _build_consts()


@jax.jit
def kernel(node_input, batch, mean_shift, affine_weight, affine_bias):
    n = node_input.shape[0]
    nb = n // BN
    x = node_input
    b = batch.astype(jnp.int32)
    b3 = b.reshape(nb, 1, BN)
    first = b3[:, 0, 0]
    last = b3[:, 0, BN - 1]
    b0a = (first // 8) * 8
    span = last - first + 1

    s_all = jnp.searchsorted(b, jnp.arange(G + 1, dtype=jnp.int32))
    s_all = s_all.astype(jnp.int32)
    gmat = s_all[(jnp.arange(32) * GPT)[:, None] + jnp.arange(GPT + 1)[None, :]]
    bnd = jnp.concatenate(
        [gmat, jnp.full((32, NLANE - GPT - 1), n, jnp.int32)],
        axis=1).reshape(-1)

    stats = _sc_stats(x, bnd)

    table = pl.pallas_call(
        _table_kernel,
        out_shape=jax.ShapeDtypeStruct((GP, TW), jnp.float32),
    )(stats, _GM, _EM, _DINV,
      affine_weight[None, :], mean_shift[None, :], affine_bias[None, :])

    out = pl.pallas_call(
        _apply_kernel,
        grid=(nb,),
        in_specs=[
            pl.BlockSpec((BN, DIM), lambda i: (i, 0)),
            pl.BlockSpec((1, 1, BN), lambda i: (i, 0, 0)),
            pl.BlockSpec(memory_space=pltpu.SMEM),
            pl.BlockSpec(memory_space=pltpu.SMEM),
            pl.BlockSpec((GP, TW), lambda i: (0, 0)),
        ],
        out_specs=pl.BlockSpec((BN, DIM), lambda i: (i, 0)),
        out_shape=jax.ShapeDtypeStruct((n, DIM), jnp.float32),
    )(x, b3, b0a, span, table)

    return out


# speedup vs baseline: 1.6126x; 1.6126x over previous
import functools

import jax
import jax.numpy as jnp
import numpy as np
from jax import lax
from jax.experimental import pallas as pl
from jax.experimental.pallas import tpu as pltpu
from jax.experimental.pallas import tpu_sc as plsc

IRREPS = [(128, 0, 1), (64, 1, -1), (32, 2, 1)]
G = 256
EPS = 1e-05
DIM = 480
NMUL = 224
NSC = 128
SW = 640
TW = 640
BN = 1000
W = 40
GP = G + W

CHUNK = 200
NLANE = 16


GPT = G // 32


def _sc_stats_body(x_hbm, bnd_hbm, o_hbm, xbuf, bndv, local):
    cid = lax.axis_index("c")
    sid = lax.axis_index("s")
    wid = sid * 2 + cid

    pltpu.sync_copy(
        bnd_hbm.at[pl.ds(pl.multiple_of(wid * NLANE, 8), NLANE)], bndv)
    bounds = bndv[...]
    r0 = bounds[0]
    r1 = bounds[GPT]

    zero16 = jnp.zeros((NLANE,), jnp.float32)
    ones16 = jnp.ones((NLANE,), jnp.float32)
    for i in range(GPT):
        for j in range(SW // NLANE):
            local[i, pl.ds(j * NLANE, NLANE)] = zero16

    start0 = pl.multiple_of((r0 // 8) * 8, 8)
    rtail = pl.multiple_of(
        jnp.maximum((r1 - CHUNK + 7) // 8 * 8, 0), 8)
    nch = (r1 - start0 + CHUNK - 1) // CHUNK

    def do_chunk(k, pend):
        cb = pl.multiple_of(jnp.minimum(start0 + k * CHUNK, rtail), 8)
        hi_abs = jnp.minimum(cb + CHUNK, r1)
        pltpu.sync_copy(x_hbm.at[pl.ds(cb, CHUNK), :], xbuf)
        pstart = jnp.maximum(pend, cb)

        nsg = NSC // NLANE
        nqg = DIM // NLANE

        for g in range(GPT):
            lo = jnp.maximum(bounds[g] - cb, pstart - cb)
            hi = jnp.maximum(jnp.minimum(bounds[g + 1], hi_abs) - cb, lo)

            def row(r, acc):
                sums, sqs = acc[:nsg], acc[nsg:]
                out = []
                for j in range(nqg):
                    v = xbuf[r, pl.ds(j * NLANE, NLANE)]
                    if j < nsg:
                        out.append((j, sums[j] + v))
                    sqs = list(sqs)
                    sqs[j] = sqs[j] + v * v
                new_sums = list(sums)
                for j, s in out:
                    new_sums[j] = s
                return tuple(new_sums) + tuple(sqs)

            acc = lax.fori_loop(lo, hi, row, (zero16,) * (nsg + nqg))
            for j in range(nsg):
                local[g, pl.ds(j * NLANE, NLANE)] += acc[j]
            for j in range(nqg):
                local[g, pl.ds(NSC + j * NLANE, NLANE)] += acc[nsg + j]
        return jnp.maximum(hi_abs, pend)

    lax.fori_loop(0, nch, do_chunk, r0)
    for g in range(GPT):
        cnt = (bounds[g + 1] - bounds[g]).astype(jnp.float32)
        local[g, pl.ds(SW - 32, NLANE)] = jnp.full((NLANE,), cnt)
    pltpu.sync_copy(local,
                    o_hbm.at[pl.ds(pl.multiple_of(wid * GPT, 8), GPT), :])


def _sc_stats(x, bnd):
    mesh = plsc.VectorSubcoreMesh(core_axis_name="c", subcore_axis_name="s")
    f = pl.kernel(
        _sc_stats_body,
        out_type=jax.ShapeDtypeStruct((G, SW), jnp.float32),
        mesh=mesh,
        scratch_types=[
            pltpu.VMEM((CHUNK, DIM), jnp.float32),
            pltpu.VMEM((NLANE,), jnp.int32),
            pltpu.VMEM((GPT, SW), jnp.float32),
        ],
    )
    return f(x, bnd)


def _table_kernel(st_ref, gm_ref, em_ref, dinv_ref, w_ref, ms_ref,
                  bias_ref, t_ref):
    st = st_ref[...]
    cnt = jnp.maximum(st[:, SW - 32:SW - 31], 1.0)
    s1 = st[:, :NSC]
    sq = st[:, NSC:NSC + DIM]
    m = s1 / cnt
    gq = jax.lax.dot_general(sq, gm_ref[...], (((1,), (0,)), ((), ())),
                             preferred_element_type=jnp.float32)
    ex2 = gq * dinv_ref[...] / cnt
    ms = ms_ref[...]
    corr = (m * m) * (ms * (2.0 - ms))
    corr_p = jnp.concatenate([corr, jnp.zeros((G, NMUL - NSC), jnp.float32)],
                             axis=1)
    fn = jax.lax.rsqrt(ex2 - corr_p + EPS) * w_ref[...]
    scale = jax.lax.dot_general(fn, em_ref[...], (((1,), (0,)), ((), ())),
                                preferred_element_type=jnp.float32)
    off_sc = bias_ref[...] - m * ms * fn[:, :NSC]
    tbl = jnp.concatenate([scale, off_sc,
                           jnp.zeros((G, TW - DIM - NSC), jnp.float32)],
                          axis=1)
    t_ref[...] = jnp.concatenate(
        [tbl, jnp.zeros((GP - G, TW), jnp.float32)], axis=0)


def _apply_kernel(x_ref, b_ref, b0_ref, sp_ref, t_ref, o_ref):
    x = x_ref[...]
    bn = x.shape[0]
    seg = b_ref[0, 0, :]
    b0a = pl.multiple_of(b0_ref[pl.program_id(0)], 8)
    span = sp_ref[pl.program_id(0)]

    def fin(so):
        o_ref[...] = jnp.concatenate(
            [x[:, :NSC] * so[:, :NSC] + so[:, DIM:DIM + NSC],
             x[:, NSC:] * so[:, NSC:DIM]], axis=1)

    @pl.when(span <= W - 8)
    def _():
        iota = jax.lax.broadcasted_iota(jnp.int32, (BN, W), 1) + b0a
        oh = (iota == seg[:, None]).astype(jnp.float32)
        fin(jax.lax.dot_general(oh, t_ref[pl.ds(b0a, W), :],
                                (((1,), (0,)), ((), ())),
                                preferred_element_type=jnp.float32))

    @pl.when(span > W - 8)
    def _():
        iota = jax.lax.broadcasted_iota(jnp.int32, (BN, G), 1)
        oh = (iota == seg[:, None]).astype(jnp.float32)
        fin(jax.lax.dot_general(oh, t_ref[pl.ds(0, G), :],
                                (((1,), (0,)), ((), ())),
                                preferred_element_type=jnp.float32))


def _build_consts():
    d_of = np.concatenate([np.full(mul, 2 * l + 1, np.float32)
                           for mul, l, p in IRREPS])
    gm = np.zeros((DIM, NMUL), np.float32)
    em = np.zeros((NMUL, DIM), np.float32)
    c = 0
    mi = 0
    for mul, l, p in IRREPS:
        d = 2 * l + 1
        for k in range(mul):
            gm[c:c + d, mi] = 1.0
            em[mi, c:c + d] = 1.0
            c += d
            mi += 1
    dinv = (1.0 / d_of)[None, :]
    return gm, em, dinv


_GM, _EM, _DINV = _build_consts()


@jax.jit
def kernel(node_input, batch, mean_shift, affine_weight, affine_bias):
    n = node_input.shape[0]
    nb = n // BN
    x = node_input
    b = batch.astype(jnp.int32)
    b3 = b.reshape(nb, 1, BN)
    first = b3[:, 0, 0]
    last = b3[:, 0, BN - 1]
    b0a = (first // 8) * 8
    span = last - first + 1

    s_all = jnp.searchsorted(b, jnp.arange(G + 1, dtype=jnp.int32))
    s_all = s_all.astype(jnp.int32)
    gmat = s_all[(jnp.arange(32) * GPT)[:, None] + jnp.arange(GPT + 1)[None, :]]
    bnd = jnp.concatenate(
        [gmat, jnp.full((32, NLANE - GPT - 1), n, jnp.int32)],
        axis=1).reshape(-1)

    stats = _sc_stats(x, bnd)

    table = pl.pallas_call(
        _table_kernel,
        out_shape=jax.ShapeDtypeStruct((GP, TW), jnp.float32),
    )(stats, _GM, _EM, _DINV,
      affine_weight[None, :], mean_shift[None, :], affine_bias[None, :])

    out = pl.pallas_call(
        _apply_kernel,
        grid=(nb,),
        in_specs=[
            pl.BlockSpec((BN, DIM), lambda i: (i, 0)),
            pl.BlockSpec((1, 1, BN), lambda i: (i, 0, 0)),
            pl.BlockSpec(memory_space=pltpu.SMEM),
            pl.BlockSpec(memory_space=pltpu.SMEM),
            pl.BlockSpec((GP, TW), lambda i: (0, 0)),
        ],
        out_specs=pl.BlockSpec((BN, DIM), lambda i: (i, 0)),
        out_shape=jax.ShapeDtypeStruct((n, DIM), jnp.float32),
    )(x, b3, b0a, span, table)

    return out


# speedup vs baseline: 1.6803x; 1.0419x over previous
import functools

import jax
import jax.numpy as jnp
import numpy as np
from jax import lax
from jax.experimental import pallas as pl
from jax.experimental.pallas import tpu as pltpu
from jax.experimental.pallas import tpu_sc as plsc

IRREPS = [(128, 0, 1), (64, 1, -1), (32, 2, 1)]
G = 256
EPS = 1e-05
DIM = 480
NMUL = 224
NSC = 128
SW = 640
TW = 640
BN = 1000
W = 40
GP = G + W

CHUNK = 104
NLANE = 16


GPT = G // 32


def _sc_stats_body(x_hbm, bnd_hbm, o_hbm, xbuf0, xbuf1, bndv, local,
                   sem0, sem1):
    cid = lax.axis_index("c")
    sid = lax.axis_index("s")
    wid = sid * 2 + cid

    pltpu.sync_copy(
        bnd_hbm.at[pl.ds(pl.multiple_of(wid * NLANE, 8), NLANE)], bndv)
    bounds = bndv[...]
    r0 = bounds[0]
    r1 = bounds[GPT]

    zero16 = jnp.zeros((NLANE,), jnp.float32)
    ones16 = jnp.ones((NLANE,), jnp.float32)
    for i in range(GPT):
        for j in range(SW // NLANE):
            local[i, pl.ds(j * NLANE, NLANE)] = zero16

    start0 = pl.multiple_of((r0 // 8) * 8, 8)
    rtail = pl.multiple_of(
        jnp.maximum((r1 - CHUNK + 7) // 8 * 8, 0), 8)
    nch = (r1 - start0 + CHUNK - 1) // CHUNK

    def cbase(k):
        return pl.multiple_of(jnp.minimum(start0 + k * CHUNK, rtail), 8)

    def issue(k, buf, sem):
        return pltpu.async_copy(x_hbm.at[pl.ds(cbase(k), CHUNK), :], buf, sem)

    issue(0, xbuf0, sem0)
    issue(1, xbuf1, sem1)

    def do_chunk(k, pend, xbuf, sem):
        cb = cbase(k)
        hi_abs = jnp.minimum(cb + CHUNK, r1)
        pltpu.make_async_copy(x_hbm.at[pl.ds(cb, CHUNK), :], xbuf, sem).wait()
        pstart = jnp.maximum(pend, cb)

        nsg = NSC // NLANE
        nqg = DIM // NLANE

        for g in range(GPT):
            lo = jnp.maximum(bounds[g] - cb, pstart - cb)
            hi = jnp.maximum(jnp.minimum(bounds[g + 1], hi_abs) - cb, lo)

            def row(r, acc):
                sums, sqs = acc[:nsg], acc[nsg:]
                out = []
                for j in range(nqg):
                    v = xbuf[r, pl.ds(j * NLANE, NLANE)]
                    if j < nsg:
                        out.append((j, sums[j] + v))
                    sqs = list(sqs)
                    sqs[j] = sqs[j] + v * v
                new_sums = list(sums)
                for j, s in out:
                    new_sums[j] = s
                return tuple(new_sums) + tuple(sqs)

            acc = lax.fori_loop(lo, hi, row, (zero16,) * (nsg + nqg))
            for j in range(nsg):
                local[g, pl.ds(j * NLANE, NLANE)] += acc[j]
            for j in range(nqg):
                local[g, pl.ds(NSC + j * NLANE, NLANE)] += acc[nsg + j]
        return jnp.maximum(hi_abs, pend)

    def pair(k2, pend):
        k = 2 * k2
        pend = do_chunk(k, pend, xbuf0, sem0)
        issue(k + 2, xbuf0, sem0)
        pend = do_chunk(k + 1, pend, xbuf1, sem1)
        issue(k + 3, xbuf1, sem1)
        return pend

    lax.fori_loop(0, (nch + 1) // 2, pair, r0)
    pltpu.make_async_copy(x_hbm.at[pl.ds(0, CHUNK), :], xbuf0, sem0).wait()
    pltpu.make_async_copy(x_hbm.at[pl.ds(0, CHUNK), :], xbuf1, sem1).wait()
    for g in range(GPT):
        cnt = (bounds[g + 1] - bounds[g]).astype(jnp.float32)
        local[g, pl.ds(SW - 32, NLANE)] = jnp.full((NLANE,), cnt)
    pltpu.sync_copy(local,
                    o_hbm.at[pl.ds(pl.multiple_of(wid * GPT, 8), GPT), :])


def _sc_stats(x, bnd):
    mesh = plsc.VectorSubcoreMesh(core_axis_name="c", subcore_axis_name="s")
    f = pl.kernel(
        _sc_stats_body,
        out_type=jax.ShapeDtypeStruct((G, SW), jnp.float32),
        mesh=mesh,
        scratch_types=[
            pltpu.VMEM((CHUNK, DIM), jnp.float32),
            pltpu.VMEM((CHUNK, DIM), jnp.float32),
            pltpu.VMEM((NLANE,), jnp.int32),
            pltpu.VMEM((GPT, SW), jnp.float32),
            pltpu.SemaphoreType.DMA,
            pltpu.SemaphoreType.DMA,
        ],
    )
    return f(x, bnd)


def _table_kernel(st_ref, gm_ref, em_ref, dinv_ref, w_ref, ms_ref,
                  bias_ref, t_ref):
    st = st_ref[...]
    cnt = jnp.maximum(st[:, SW - 32:SW - 31], 1.0)
    s1 = st[:, :NSC]
    sq = st[:, NSC:NSC + DIM]
    m = s1 / cnt
    gq = jax.lax.dot_general(sq, gm_ref[...], (((1,), (0,)), ((), ())),
                             preferred_element_type=jnp.float32)
    ex2 = gq * dinv_ref[...] / cnt
    ms = ms_ref[...]
    corr = (m * m) * (ms * (2.0 - ms))
    corr_p = jnp.concatenate([corr, jnp.zeros((G, NMUL - NSC), jnp.float32)],
                             axis=1)
    fn = jax.lax.rsqrt(ex2 - corr_p + EPS) * w_ref[...]
    scale = jax.lax.dot_general(fn, em_ref[...], (((1,), (0,)), ((), ())),
                                preferred_element_type=jnp.float32)
    off_sc = bias_ref[...] - m * ms * fn[:, :NSC]
    tbl = jnp.concatenate([scale, off_sc,
                           jnp.zeros((G, TW - DIM - NSC), jnp.float32)],
                          axis=1)
    t_ref[...] = jnp.concatenate(
        [tbl, jnp.zeros((GP - G, TW), jnp.float32)], axis=0)


def _apply_kernel(x_ref, b_ref, b0_ref, sp_ref, t_ref, o_ref):
    x = x_ref[...]
    bn = x.shape[0]
    seg = b_ref[0, 0, :]
    b0a = pl.multiple_of(b0_ref[pl.program_id(0)], 8)
    span = sp_ref[pl.program_id(0)]

    def fin(so):
        o_ref[...] = jnp.concatenate(
            [x[:, :NSC] * so[:, :NSC] + so[:, DIM:DIM + NSC],
             x[:, NSC:] * so[:, NSC:DIM]], axis=1)

    @pl.when(span <= W - 8)
    def _():
        iota = jax.lax.broadcasted_iota(jnp.int32, (BN, W), 1) + b0a
        oh = (iota == seg[:, None]).astype(jnp.float32)
        fin(jax.lax.dot_general(oh, t_ref[pl.ds(b0a, W), :],
                                (((1,), (0,)), ((), ())),
                                preferred_element_type=jnp.float32))

    @pl.when(span > W - 8)
    def _():
        iota = jax.lax.broadcasted_iota(jnp.int32, (BN, G), 1)
        oh = (iota == seg[:, None]).astype(jnp.float32)
        fin(jax.lax.dot_general(oh, t_ref[pl.ds(0, G), :],
                                (((1,), (0,)), ((), ())),
                                preferred_element_type=jnp.float32))


def _build_consts():
    d_of = np.concatenate([np.full(mul, 2 * l + 1, np.float32)
                           for mul, l, p in IRREPS])
    gm = np.zeros((DIM, NMUL), np.float32)
    em = np.zeros((NMUL, DIM), np.float32)
    c = 0
    mi = 0
    for mul, l, p in IRREPS:
        d = 2 * l + 1
        for k in range(mul):
            gm[c:c + d, mi] = 1.0
            em[mi, c:c + d] = 1.0
            c += d
            mi += 1
    dinv = (1.0 / d_of)[None, :]
    return gm, em, dinv


_GM, _EM, _DINV = _build_consts()


@jax.jit
def kernel(node_input, batch, mean_shift, affine_weight, affine_bias):
    n = node_input.shape[0]
    nb = n // BN
    x = node_input
    b = batch.astype(jnp.int32)
    b3 = b.reshape(nb, 1, BN)
    first = b3[:, 0, 0]
    last = b3[:, 0, BN - 1]
    b0a = (first // 8) * 8
    span = last - first + 1

    s_all = jnp.searchsorted(b, jnp.arange(G + 1, dtype=jnp.int32))
    s_all = s_all.astype(jnp.int32)
    gmat = s_all[(jnp.arange(32) * GPT)[:, None] + jnp.arange(GPT + 1)[None, :]]
    bnd = jnp.concatenate(
        [gmat, jnp.full((32, NLANE - GPT - 1), n, jnp.int32)],
        axis=1).reshape(-1)

    stats = _sc_stats(x, bnd)

    table = pl.pallas_call(
        _table_kernel,
        out_shape=jax.ShapeDtypeStruct((GP, TW), jnp.float32),
    )(stats, _GM, _EM, _DINV,
      affine_weight[None, :], mean_shift[None, :], affine_bias[None, :])

    out = pl.pallas_call(
        _apply_kernel,
        grid=(nb,),
        in_specs=[
            pl.BlockSpec((BN, DIM), lambda i: (i, 0)),
            pl.BlockSpec((1, 1, BN), lambda i: (i, 0, 0)),
            pl.BlockSpec(memory_space=pltpu.SMEM),
            pl.BlockSpec(memory_space=pltpu.SMEM),
            pl.BlockSpec((GP, TW), lambda i: (0, 0)),
        ],
        out_specs=pl.BlockSpec((BN, DIM), lambda i: (i, 0)),
        out_shape=jax.ShapeDtypeStruct((n, DIM), jnp.float32),
    )(x, b3, b0a, span, table)

    return out


# speedup vs baseline: 1.7439x; 1.0379x over previous
import functools

import jax
import jax.numpy as jnp
import numpy as np
from jax import lax
from jax.experimental import pallas as pl
from jax.experimental.pallas import tpu as pltpu
from jax.experimental.pallas import tpu_sc as plsc

IRREPS = [(128, 0, 1), (64, 1, -1), (32, 2, 1)]
G = 256
EPS = 1e-05
DIM = 480
NMUL = 224
NSC = 128
SW = 640
TW = 640
BN = 2000
W = 40
GP = G + W

CHUNK = 104
NLANE = 16


GPT = G // 32


def _sc_stats_body(x_hbm, bnd_hbm, o_hbm, xbuf0, xbuf1, bndv, local,
                   sem0, sem1):
    cid = lax.axis_index("c")
    sid = lax.axis_index("s")
    wid = sid * 2 + cid

    pltpu.sync_copy(
        bnd_hbm.at[pl.ds(pl.multiple_of(wid * NLANE, 8), NLANE)], bndv)
    bounds = bndv[...]
    r0 = bounds[0]
    r1 = bounds[GPT]

    zero16 = jnp.zeros((NLANE,), jnp.float32)
    ones16 = jnp.ones((NLANE,), jnp.float32)
    for i in range(GPT):
        for j in range(SW // NLANE):
            local[i, pl.ds(j * NLANE, NLANE)] = zero16

    start0 = pl.multiple_of((r0 // 8) * 8, 8)
    rtail = pl.multiple_of(
        jnp.maximum((r1 - CHUNK + 7) // 8 * 8, 0), 8)
    nch = (r1 - start0 + CHUNK - 1) // CHUNK

    def cbase(k):
        return pl.multiple_of(jnp.minimum(start0 + k * CHUNK, rtail), 8)

    def issue(k, buf, sem):
        return pltpu.async_copy(x_hbm.at[pl.ds(cbase(k), CHUNK), :], buf, sem)

    issue(0, xbuf0, sem0)
    issue(1, xbuf1, sem1)

    def do_chunk(k, pend, xbuf, sem):
        cb = cbase(k)
        hi_abs = jnp.minimum(cb + CHUNK, r1)
        pltpu.make_async_copy(x_hbm.at[pl.ds(cb, CHUNK), :], xbuf, sem).wait()
        pstart = jnp.maximum(pend, cb)

        nsg = NSC // NLANE
        nqg = DIM // NLANE

        for g in range(GPT):
            lo = jnp.maximum(bounds[g] - cb, pstart - cb)
            hi = jnp.maximum(jnp.minimum(bounds[g + 1], hi_abs) - cb, lo)

            def row(r, acc):
                sums, sqs = acc[:nsg], acc[nsg:]
                out = []
                for j in range(nqg):
                    v = xbuf[r, pl.ds(j * NLANE, NLANE)]
                    if j < nsg:
                        out.append((j, sums[j] + v))
                    sqs = list(sqs)
                    sqs[j] = sqs[j] + v * v
                new_sums = list(sums)
                for j, s in out:
                    new_sums[j] = s
                return tuple(new_sums) + tuple(sqs)

            acc = lax.fori_loop(lo, hi, row, (zero16,) * (nsg + nqg))
            for j in range(nsg):
                local[g, pl.ds(j * NLANE, NLANE)] += acc[j]
            for j in range(nqg):
                local[g, pl.ds(NSC + j * NLANE, NLANE)] += acc[nsg + j]
        return jnp.maximum(hi_abs, pend)

    def pair(k2, pend):
        k = 2 * k2
        pend = do_chunk(k, pend, xbuf0, sem0)
        issue(k + 2, xbuf0, sem0)
        pend = do_chunk(k + 1, pend, xbuf1, sem1)
        issue(k + 3, xbuf1, sem1)
        return pend

    lax.fori_loop(0, (nch + 1) // 2, pair, r0)
    pltpu.make_async_copy(x_hbm.at[pl.ds(0, CHUNK), :], xbuf0, sem0).wait()
    pltpu.make_async_copy(x_hbm.at[pl.ds(0, CHUNK), :], xbuf1, sem1).wait()
    for g in range(GPT):
        cnt = (bounds[g + 1] - bounds[g]).astype(jnp.float32)
        local[g, pl.ds(SW - 32, NLANE)] = jnp.full((NLANE,), cnt)
    pltpu.sync_copy(local,
                    o_hbm.at[pl.ds(pl.multiple_of(wid * GPT, 8), GPT), :])


def _sc_stats(x, bnd):
    mesh = plsc.VectorSubcoreMesh(core_axis_name="c", subcore_axis_name="s")
    f = pl.kernel(
        _sc_stats_body,
        out_type=jax.ShapeDtypeStruct((G, SW), jnp.float32),
        mesh=mesh,
        scratch_types=[
            pltpu.VMEM((CHUNK, DIM), jnp.float32),
            pltpu.VMEM((CHUNK, DIM), jnp.float32),
            pltpu.VMEM((NLANE,), jnp.int32),
            pltpu.VMEM((GPT, SW), jnp.float32),
            pltpu.SemaphoreType.DMA,
            pltpu.SemaphoreType.DMA,
        ],
    )
    return f(x, bnd)


def _table_kernel(st_ref, gm_ref, em_ref, dinv_ref, w_ref, ms_ref,
                  bias_ref, t_ref):
    st = st_ref[...]
    cnt = jnp.maximum(st[:, SW - 32:SW - 31], 1.0)
    s1 = st[:, :NSC]
    sq = st[:, NSC:NSC + DIM]
    m = s1 / cnt
    gq = jax.lax.dot_general(sq, gm_ref[...], (((1,), (0,)), ((), ())),
                             preferred_element_type=jnp.float32)
    ex2 = gq * dinv_ref[...] / cnt
    ms = ms_ref[...]
    corr = (m * m) * (ms * (2.0 - ms))
    corr_p = jnp.concatenate([corr, jnp.zeros((G, NMUL - NSC), jnp.float32)],
                             axis=1)
    fn = jax.lax.rsqrt(ex2 - corr_p + EPS) * w_ref[...]
    scale = jax.lax.dot_general(fn, em_ref[...], (((1,), (0,)), ((), ())),
                                preferred_element_type=jnp.float32)
    off_sc = bias_ref[...] - m * ms * fn[:, :NSC]
    tbl = jnp.concatenate([scale, off_sc,
                           jnp.zeros((G, TW - DIM - NSC), jnp.float32)],
                          axis=1)
    t_ref[...] = jnp.concatenate(
        [tbl, jnp.zeros((GP - G, TW), jnp.float32)], axis=0)


def _apply_kernel(x_ref, b_ref, b0_ref, sp_ref, t_ref, o_ref):
    x = x_ref[...]
    bn = x.shape[0]
    seg = b_ref[0, 0, :]
    b0a = pl.multiple_of(b0_ref[pl.program_id(0)], 8)
    span = sp_ref[pl.program_id(0)]

    def fin(so):
        o_ref[...] = jnp.concatenate(
            [x[:, :NSC] * so[:, :NSC] + so[:, DIM:DIM + NSC],
             x[:, NSC:] * so[:, NSC:DIM]], axis=1)

    @pl.when(span <= W - 8)
    def _():
        iota = jax.lax.broadcasted_iota(jnp.int32, (BN, W), 1) + b0a
        oh = (iota == seg[:, None]).astype(jnp.float32)
        fin(jax.lax.dot_general(oh, t_ref[pl.ds(b0a, W), :],
                                (((1,), (0,)), ((), ())),
                                preferred_element_type=jnp.float32))

    @pl.when(span > W - 8)
    def _():
        iota = jax.lax.broadcasted_iota(jnp.int32, (BN, G), 1)
        oh = (iota == seg[:, None]).astype(jnp.float32)
        fin(jax.lax.dot_general(oh, t_ref[pl.ds(0, G), :],
                                (((1,), (0,)), ((), ())),
                                preferred_element_type=jnp.float32))


def _build_consts():
    d_of = np.concatenate([np.full(mul, 2 * l + 1, np.float32)
                           for mul, l, p in IRREPS])
    gm = np.zeros((DIM, NMUL), np.float32)
    em = np.zeros((NMUL, DIM), np.float32)
    c = 0
    mi = 0
    for mul, l, p in IRREPS:
        d = 2 * l + 1
        for k in range(mul):
            gm[c:c + d, mi] = 1.0
            em[mi, c:c + d] = 1.0
            c += d
            mi += 1
    dinv = (1.0 / d_of)[None, :]
    return gm, em, dinv


_GM, _EM, _DINV = _build_consts()


@jax.jit
def kernel(node_input, batch, mean_shift, affine_weight, affine_bias):
    n = node_input.shape[0]
    nb = n // BN
    x = node_input
    b = batch.astype(jnp.int32)
    b3 = b.reshape(nb, 1, BN)
    first = b3[:, 0, 0]
    last = b3[:, 0, BN - 1]
    b0a = (first // 8) * 8
    span = last - first + 1

    s_all = jnp.searchsorted(b, jnp.arange(G + 1, dtype=jnp.int32))
    s_all = s_all.astype(jnp.int32)
    gmat = s_all[(jnp.arange(32) * GPT)[:, None] + jnp.arange(GPT + 1)[None, :]]
    bnd = jnp.concatenate(
        [gmat, jnp.full((32, NLANE - GPT - 1), n, jnp.int32)],
        axis=1).reshape(-1)

    stats = _sc_stats(x, bnd)

    table = pl.pallas_call(
        _table_kernel,
        out_shape=jax.ShapeDtypeStruct((GP, TW), jnp.float32),
    )(stats, _GM, _EM, _DINV,
      affine_weight[None, :], mean_shift[None, :], affine_bias[None, :])

    out = pl.pallas_call(
        _apply_kernel,
        grid=(nb,),
        in_specs=[
            pl.BlockSpec((BN, DIM), lambda i: (i, 0)),
            pl.BlockSpec((1, 1, BN), lambda i: (i, 0, 0)),
            pl.BlockSpec(memory_space=pltpu.SMEM),
            pl.BlockSpec(memory_space=pltpu.SMEM),
            pl.BlockSpec((GP, TW), lambda i: (0, 0)),
        ],
        out_specs=pl.BlockSpec((BN, DIM), lambda i: (i, 0)),
        out_shape=jax.ShapeDtypeStruct((n, DIM), jnp.float32),
    )(x, b3, b0a, span, table)

    return out
